# baseline (speedup 1.0000x reference)
import functools
import math

import jax
import jax.numpy as jnp
from jax import lax
from jax.experimental import pallas as pl
from jax.experimental.pallas import tpu as pltpu
from jax.experimental.pallas import tpu_sc as plsc

D_MODEL = 32
BATCH = 16384
HIST = 50
SCALE = math.sqrt(D_MODEL)

_info = plsc.get_sparse_core_info()
NC = _info.num_cores
NS = _info.num_subcores
NW = NC * NS
B_PER_W = BATCH // NW
NB = 16
N_CHUNKS = B_PER_W // NB
HIST_PAD = 56


def _body(w_hbm, x_hbm, out_hbm, xb_v, rows_v, sem):
    wid = lax.axis_index("s") * NC + lax.axis_index("c")
    b_base = wid * B_PER_W

    def chunk_body(c, carry):
        b0 = b_base + c * NB
        pltpu.sync_copy(x_hbm.at[pl.ds(b0, NB), :], xb_v)
        descs = [
            pltpu.async_copy(
                w_hbm.at[xb_v.at[i, pl.ds(0, HIST_PAD)]], rows_v.at[i], sem
            )
            for i in range(NB)
        ]
        for d in descs:
            d.wait()

        def scale_b(bi, carry2):
            def scale_k(k, carry3):
                h = k >> 1
                off = (k & 1) * 16
                rows_v[bi, h, pl.ds(off, 16)] = (
                    rows_v[bi, h, pl.ds(off, 16)] * SCALE
                )
                return carry3

            return lax.fori_loop(0, 2 * HIST, scale_k, carry2, unroll=4)

        lax.fori_loop(0, NB, scale_b, 0)
        wdescs = [
            pltpu.async_copy(
                rows_v.at[i, pl.ds(0, HIST)], out_hbm.at[b0 + i], sem
            )
            for i in range(NB)
        ]
        for d in wdescs:
            d.wait()
        return carry

    lax.fori_loop(0, N_CHUNKS, chunk_body, 0)


_sc_kernel = functools.partial(
    pl.kernel,
    out_type=jax.ShapeDtypeStruct((BATCH, HIST, D_MODEL), jnp.float32),
    mesh=plsc.VectorSubcoreMesh(core_axis_name="c", subcore_axis_name="s"),
    scratch_types=[
        pltpu.VMEM((NB, 128), jnp.int32),
        pltpu.VMEM((NB, HIST_PAD, D_MODEL), jnp.float32),
        pltpu.SemaphoreType.DMA,
    ],
    compiler_params=pltpu.CompilerParams(use_tc_tiling_on_sc=False),
)(_body)


@jax.jit
def kernel(x, weight):
    xp = jnp.pad(x, ((0, 0), (0, 128 - HIST)))
    return _sc_kernel(weight, xp)


# speedup vs baseline: 1.3920x; 1.3920x over previous
import functools
import math

import jax
import jax.numpy as jnp
from jax import lax
from jax.experimental import pallas as pl
from jax.experimental.pallas import tpu as pltpu
from jax.experimental.pallas import tpu_sc as plsc

D_MODEL = 32
BATCH = 16384
HIST = 50
SCALE = math.sqrt(D_MODEL)

_info = plsc.get_sparse_core_info()
NC = _info.num_cores
NS = _info.num_subcores
NW = NC * NS
B_PER_W = BATCH // NW
FLAT_ROWS = BATCH * HIST * D_MODEL // 128

_MESH = plsc.VectorSubcoreMesh(core_axis_name="c", subcore_axis_name="s")

NB = 16
N_CHUNKS = B_PER_W // NB
ROWS_PER_CHUNK = NB * HIST
FLAT_PER_CHUNK = ROWS_PER_CHUNK * D_MODEL // 128


def _gather_body(w_hbm, x_hbm, outf_hbm, xb_v, rows_v, pack_v, sem):
    wid = lax.axis_index("s") * NC + lax.axis_index("c")
    b_base = wid * B_PER_W

    def chunk_body(c, carry):
        b0 = b_base + c * NB
        pltpu.sync_copy(x_hbm.at[pl.ds(b0, NB), :], xb_v)
        descs = [
            pltpu.async_copy(
                w_hbm.at[xb_v.at[i, :]],
                rows_v.at[pl.ds(i * HIST, HIST)],
                sem,
            )
            for i in range(NB)
        ]
        for d in descs:
            d.wait()

        def scale_k(k, carry2):
            pack_v[k >> 3, pl.ds((k & 7) * 16, 16)] = (
                rows_v[k >> 1, pl.ds((k & 1) * 16, 16)] * SCALE
            )
            return carry2

        lax.fori_loop(0, 2 * ROWS_PER_CHUNK, scale_k, 0, unroll=8)
        r0 = (b0 * HIST * D_MODEL) // 128
        pltpu.sync_copy(pack_v, outf_hbm.at[pl.ds(r0, FLAT_PER_CHUNK)])
        return carry

    lax.fori_loop(0, N_CHUNKS, chunk_body, 0)


_gather_kernel = functools.partial(
    pl.kernel,
    out_type=jax.ShapeDtypeStruct((FLAT_ROWS, 128), jnp.float32),
    mesh=_MESH,
    scratch_types=[
        pltpu.VMEM((NB, HIST), jnp.int32),
        pltpu.VMEM((ROWS_PER_CHUNK, D_MODEL), jnp.float32),
        pltpu.VMEM((FLAT_PER_CHUNK, 128), jnp.float32),
        pltpu.SemaphoreType.DMA,
    ],
    compiler_params=pltpu.CompilerParams(use_tc_tiling_on_sc=False),
)(_gather_body)

NBF = 16
NF_CHUNKS = B_PER_W // NBF
ROWS_F = NBF * HIST
FLAT_F = ROWS_F * D_MODEL // 128
NBH = NBF // 2


def _format_body(flat_hbm, out_hbm, stage128_v, stage32_v):
    wid = lax.axis_index("s") * NC + lax.axis_index("c")
    b_base = wid * B_PER_W

    def chunk_body(c, carry):
        b0 = b_base + c * NBF
        r0 = pl.multiple_of((b0 * HIST * D_MODEL) // 128, 8)
        pltpu.sync_copy(flat_hbm.at[pl.ds(r0, FLAT_F)], stage128_v)
        for half in range(2):

            def repack_b(b, carry2, half=half):
                def repack_k(k, carry3):
                    kg = (half * NBH + b) * (2 * HIST) + k
                    stage32_v[b, k >> 1, pl.ds((k & 1) * 16, 16)] = (
                        stage128_v[kg >> 3, pl.ds((kg & 7) * 16, 16)]
                    )
                    return carry3

                return lax.fori_loop(0, 2 * HIST, repack_k, carry2, unroll=4)

            lax.fori_loop(0, NBH, repack_b, 0)
            pltpu.sync_copy(
                stage32_v, out_hbm.at[pl.ds(b0 + half * NBH, NBH)]
            )
        return carry

    lax.fori_loop(0, NF_CHUNKS, chunk_body, 0)


_format_kernel = functools.partial(
    pl.kernel,
    out_type=jax.ShapeDtypeStruct((BATCH, HIST, D_MODEL), jnp.float32),
    mesh=_MESH,
    scratch_types=[
        pltpu.VMEM((FLAT_F, 128), jnp.float32),
        pltpu.VMEM((NBH, HIST, D_MODEL), jnp.float32),
    ],
    compiler_params=pltpu.CompilerParams(use_tc_tiling_on_sc=True),
)(_format_body)


@jax.jit
def kernel(x, weight):
    flat = _gather_kernel(weight, x)
    return _format_kernel(flat)


# speedup vs baseline: 1.9769x; 1.4202x over previous
import functools
import math

import jax
import jax.numpy as jnp
from jax import lax
from jax.experimental import pallas as pl
from jax.experimental.pallas import tpu as pltpu
from jax.experimental.pallas import tpu_sc as plsc

D_MODEL = 32
BATCH = 16384
HIST = 50
SCALE = math.sqrt(D_MODEL)

_info = plsc.get_sparse_core_info()
NC = _info.num_cores
NS = _info.num_subcores
NW = NC * NS
B_PER_W = BATCH // NW
NB = 16
N_CHUNKS = B_PER_W // NB


def _body(w_hbm, x_hbm, out_hbm, xb_v, rows_v, sems):
    wid = lax.axis_index("s") * NC + lax.axis_index("c")
    b_base = wid * B_PER_W

    def stage_in(c, buf):
        b0 = b_base + c * NB
        pltpu.sync_copy(x_hbm.at[pl.ds(b0, NB), :], xb_v.at[buf])
        for i in range(NB):
            pltpu.async_copy(
                w_hbm.at[xb_v.at[buf, i, :]],
                rows_v.at[buf, i],
                sems.at[buf],
            )

    def drain(buf):
        for i in range(NB):
            pltpu.make_async_copy(
                w_hbm.at[xb_v.at[buf, i, :]],
                rows_v.at[buf, i],
                sems.at[buf],
            ).wait()

    def finish(c, buf):
        b0 = b_base + c * NB

        def scale_b(bi, carry2):
            def scale_k(k, carry3):
                rows_v[buf, bi, k >> 1, pl.ds((k & 1) * 16, 16)] = (
                    rows_v[buf, bi, k >> 1, pl.ds((k & 1) * 16, 16)] * SCALE
                )
                return carry3

            return lax.fori_loop(0, 2 * HIST, scale_k, carry2, unroll=4)

        lax.fori_loop(0, NB, scale_b, 0)
        pltpu.sync_copy(rows_v.at[buf], out_hbm.at[pl.ds(b0, NB)])

    stage_in(0, 0)

    def pair_body(p, carry):
        c0 = 2 * p
        stage_in(c0 + 1, 1)
        drain(0)
        finish(c0, 0)

        @pl.when(c0 + 2 < N_CHUNKS)
        def _():
            stage_in(c0 + 2, 0)

        drain(1)
        finish(c0 + 1, 1)
        return carry

    lax.fori_loop(0, N_CHUNKS // 2, pair_body, 0)


_sc_kernel = functools.partial(
    pl.kernel,
    out_type=jax.ShapeDtypeStruct((BATCH, HIST, D_MODEL), jnp.float32),
    mesh=plsc.VectorSubcoreMesh(core_axis_name="c", subcore_axis_name="s"),
    scratch_types=[
        pltpu.VMEM((2, NB, HIST), jnp.int32),
        pltpu.VMEM((2, NB, HIST, D_MODEL), jnp.float32),
        pltpu.SemaphoreType.DMA((2,)),
    ],
    compiler_params=pltpu.CompilerParams(use_tc_tiling_on_sc=False),
)(_body)


@jax.jit
def kernel(x, weight):
    return _sc_kernel(weight, x)
